# baseline (speedup 1.0000x reference)
import functools

import jax
import jax.numpy as jnp
from jax import lax
from jax.experimental import pallas as pl
from jax.experimental.pallas import tpu as pltpu
from jax.experimental.pallas import tpu_sc as plsc

N_NODES = 5000
N_EDGES = 320000
DIM = 128
EPS = 1e-5

NS = 16
PAD = 5120
K = 128
CHUNKS = 158
EDGES_PER_TILE = CHUNKS * K
E_PAD = NS * EDGES_PER_TILE - N_EDGES
DUMP = 5100
OUT_STRIDE = 312

_mesh = plsc.VectorSubcoreMesh(core_axis_name="c", subcore_axis_name="s")


@functools.partial(
    pl.kernel,
    out_type=(
        jax.ShapeDtypeStruct((N_NODES, DIM), jnp.float32),
        jax.ShapeDtypeStruct((NS, PAD), jnp.float32),
        jax.ShapeDtypeStruct((N_NODES, DIM), jnp.float32),
        jax.ShapeDtypeStruct((NS, PAD), jnp.float32),
    ),
    mesh=_mesh,
    compiler_params=pltpu.CompilerParams(needs_layout_passes=False),
    scratch_types=[
        pltpu.VMEM_SHARED((PAD, DIM), jnp.float32),
    ] + [
        s
        for _ in range(4)
        for s in (pltpu.VMEM((K,), jnp.int32),
                  pltpu.VMEM((K,), jnp.int32),
                  pltpu.VMEM((K, DIM), jnp.float32),
                  pltpu.SemaphoreType.DMA,
                  pltpu.SemaphoreType.DMA,
                  pltpu.SemaphoreType.DMA)
    ] + [
        pltpu.VMEM((PAD,), jnp.float32),
    ],
)
def _sc_aggregate(uf_hbm, if_hbm, ui_src, ui_dst, iu_src, iu_dst, zero_f,
                  item_sum, item_deg, user_sum, user_deg,
                  acc_sh, *rest):
  sets = tuple(rest[6 * x:6 * x + 6] for x in range(4))
  degl = rest[24]
  cid = lax.axis_index("c")
  sid = lax.axis_index("s")
  row0 = sid * (PAD // NS)
  edge0 = sid * EDGES_PER_TILE
  zeros16 = jnp.zeros((16,), jnp.float32)
  ones16 = jnp.ones((16,), jnp.float32)

  pltpu.sync_copy(zero_f, acc_sh.at[pl.ds(row0, PAD // NS)])

  def zrow(i, _):
    degl[pl.ds(i * 16, 16)] = zeros16
    return ()
  lax.fori_loop(0, PAD // 16, zrow, ())
  plsc.subcore_barrier()

  def run(feat_hbm, src_hbm, dst_hbm, deg_out):
    def fire_idx(c, s):
      off = edge0 + c * K
      pltpu.async_copy(src_hbm.at[pl.ds(off, K)], s[0], s[3])
      pltpu.async_copy(dst_hbm.at[pl.ds(off, K)], s[1], s[3])

    def wait_idx(c, s):
      off = edge0 + c * K
      pltpu.make_async_copy(src_hbm.at[pl.ds(off, K)], s[0], s[3]).wait()
      pltpu.make_async_copy(dst_hbm.at[pl.ds(off, K)], s[1], s[3]).wait()

    def fire_gather(s):
      pltpu.async_copy(feat_hbm.at[s[0]], s[2], s[4])

    def wait_gather(s):
      pltpu.make_async_copy(feat_hbm.at[s[0]], s[2], s[4]).wait()

    def fire_scatter(s):
      pltpu.async_copy(s[2], acc_sh.at[s[1]], s[5], add=True)

    def wait_scatter(s):
      pltpu.make_async_copy(s[2], acc_sh.at[s[1]], s[5]).wait()

    def anchor(c, sk, drain=True, pf_idx=True, pf_gather=True):
      s = sets[sk]
      wait_gather(s)
      for i in range(K // 16):
        plsc.addupdate_scatter(degl, [s[1][pl.ds(i * 16, 16)]], ones16)
      fire_scatter(s)
      if pf_idx:
        y = sets[(sk + 2) % 4]
        if drain:
          wait_scatter(y)
        fire_idx(c + 2, y)
      if pf_gather:
        z = sets[(sk + 1) % 4]
        wait_idx(c + 1, z)
        fire_gather(z)

    fire_idx(0, sets[0])
    fire_idx(1, sets[1])
    wait_idx(0, sets[0])
    fire_gather(sets[0])
    anchor(0, 0, drain=False)
    anchor(1, 1, drain=False)

    def step(i, _):
      c0 = 4 * i + 2
      for k in range(4):
        anchor(c0 + k, (2 + k) % 4)
      return ()
    lax.fori_loop(0, (CHUNKS - 6) // 4, step, ())

    anchor(CHUNKS - 4, (CHUNKS - 4) % 4)
    anchor(CHUNKS - 3, (CHUNKS - 3) % 4)
    anchor(CHUNKS - 2, (CHUNKS - 2) % 4, pf_idx=False)
    anchor(CHUNKS - 1, (CHUNKS - 1) % 4, pf_idx=False, pf_gather=False)
    for k in range(4):
      wait_scatter(sets[(CHUNKS - 4 + k) % 4])

    pltpu.sync_copy(degl, deg_out.at[sid])

  @pl.when(cid == 0)
  def _():
    run(uf_hbm, ui_src, ui_dst, item_deg)

  @pl.when(cid == 1)
  def _():
    run(if_hbm, iu_src, iu_dst, user_deg)

  plsc.subcore_barrier()

  out0 = sid * OUT_STRIDE

  @pl.when(cid == 0)
  def _():
    pltpu.sync_copy(acc_sh.at[pl.ds(out0, 320)], item_sum.at[pl.ds(out0, 320)])

  @pl.when(cid == 1)
  def _():
    pltpu.sync_copy(acc_sh.at[pl.ds(out0, 320)], user_sum.at[pl.ds(out0, 320)])


def _tc_head_body(feat, msum, mdeg, ones_c, w_t, b, gamma, beta, out):
  d = lax.dot_general(mdeg[...], ones_c[...], (((0,), (0,)), ((), ())),
                      preferred_element_type=jnp.float32)
  r = 1.0 / jnp.maximum(d, 1.0)
  x = feat[...] + msum[...] * r
  h = jnp.dot(x, w_t[...], preferred_element_type=jnp.float32) + b[...]
  h = jnp.maximum(h, 0.0)
  mu = jnp.mean(h, axis=1, keepdims=True)
  var = jnp.mean((h - mu) ** 2, axis=1, keepdims=True)
  out[...] = (h - mu) * lax.rsqrt(var + EPS) * gamma[...] + beta[...]


def _tc_head(feat, msum, mdeg, ones_c, w_t, b, gamma, beta):
  bm = 1024
  grid = PAD // bm
  return pl.pallas_call(
      _tc_head_body,
      grid=(grid,),
      in_specs=[
          pl.BlockSpec((bm, DIM), lambda i: (i, 0)),
          pl.BlockSpec((bm, DIM), lambda i: (i, 0)),
          pl.BlockSpec((NS, bm), lambda i: (0, i)),
          pl.BlockSpec((NS, 1), lambda i: (0, 0)),
          pl.BlockSpec((DIM, DIM), lambda i: (0, 0)),
          pl.BlockSpec((1, DIM), lambda i: (0, 0)),
          pl.BlockSpec((1, DIM), lambda i: (0, 0)),
          pl.BlockSpec((1, DIM), lambda i: (0, 0)),
      ],
      out_specs=pl.BlockSpec((bm, DIM), lambda i: (i, 0)),
      out_shape=jax.ShapeDtypeStruct((N_NODES, DIM), jnp.float32),
  )(feat, msum, mdeg, ones_c, w_t, b, gamma, beta)


@jax.jit
def kernel(user_features, item_features, user_item_edge_index,
           item_user_edge_index, Wu, bu, Wi, bi, gamma, beta):
  src_pad = jnp.zeros((E_PAD,), jnp.int32)
  dst_pad = jnp.full((E_PAD,), DUMP, jnp.int32)
  ui_src = jnp.concatenate([user_item_edge_index[0].astype(jnp.int32), src_pad])
  ui_dst = jnp.concatenate([user_item_edge_index[1].astype(jnp.int32), dst_pad])
  iu_src = jnp.concatenate([item_user_edge_index[0].astype(jnp.int32), src_pad])
  iu_dst = jnp.concatenate([item_user_edge_index[1].astype(jnp.int32), dst_pad])
  zero_f = jnp.zeros((PAD // NS, DIM), jnp.float32)

  item_sum, item_deg, user_sum, user_deg = _sc_aggregate(
      user_features, item_features, ui_src, ui_dst, iu_src, iu_dst, zero_f)

  ones_c = jnp.ones((NS, 1), jnp.float32)
  g = gamma.reshape(1, DIM)
  bt = beta.reshape(1, DIM)
  user_new = _tc_head(user_features, user_sum, user_deg, ones_c,
                      Wu.T, bu.reshape(1, DIM), g, bt)
  item_new = _tc_head(item_features, item_sum, item_deg, ones_c,
                      Wi.T, bi.reshape(1, DIM), g, bt)
  return (user_new, item_new)


# speedup vs baseline: 1.3602x; 1.3602x over previous
import functools

import jax
import jax.numpy as jnp
from jax import lax
from jax.experimental import pallas as pl
from jax.experimental.pallas import tpu as pltpu
from jax.experimental.pallas import tpu_sc as plsc

N_NODES = 5000
N_EDGES = 320000
DIM = 128
EPS = 1e-5

NS = 16
PAD = 5120
K = 80
EDGES_PER_TILE = N_EDGES // NS
CHUNKS = EDGES_PER_TILE // K
OUT_STRIDE = 312

_mesh = plsc.VectorSubcoreMesh(core_axis_name="c", subcore_axis_name="s")


@functools.partial(
    pl.kernel,
    out_type=(
        jax.ShapeDtypeStruct((N_NODES, DIM), jnp.float32),
        jax.ShapeDtypeStruct((NS, PAD), jnp.float32),
        jax.ShapeDtypeStruct((N_NODES, DIM), jnp.float32),
        jax.ShapeDtypeStruct((NS, PAD), jnp.float32),
    ),
    mesh=_mesh,
    compiler_params=pltpu.CompilerParams(needs_layout_passes=False),
    scratch_types=[
        pltpu.VMEM_SHARED((PAD, DIM), jnp.float32),
    ] + [
        s
        for _ in range(4)
        for s in (pltpu.VMEM((2, K), jnp.int32),
                  pltpu.VMEM((K, DIM), jnp.float32),
                  pltpu.SemaphoreType.DMA,
                  pltpu.SemaphoreType.DMA,
                  pltpu.SemaphoreType.DMA)
    ] + [
        pltpu.VMEM((PAD,), jnp.float32),
    ],
)
def _sc_aggregate(uf_hbm, if_hbm, ui_edge, iu_edge, zero_f,
                  item_sum, item_deg, user_sum, user_deg,
                  acc_sh, *rest):
  sets = tuple(rest[5 * x:5 * x + 5] for x in range(4))
  degl = rest[20]
  cid = lax.axis_index("c")
  sid = lax.axis_index("s")
  row0 = sid * (PAD // NS)
  edge0 = sid * EDGES_PER_TILE
  zeros16 = jnp.zeros((16,), jnp.float32)
  ones16 = jnp.ones((16,), jnp.float32)

  pltpu.sync_copy(zero_f, acc_sh.at[pl.ds(row0, PAD // NS)])

  def zrow(i, _):
    degl[pl.ds(i * 16, 16)] = zeros16
    return ()
  lax.fori_loop(0, PAD // 16, zrow, ())
  plsc.subcore_barrier()

  def run(feat_hbm, edge_hbm, deg_out):
    def fire_idx(c, s):
      pltpu.async_copy(edge_hbm.at[sid * CHUNKS + c], s[0], s[2])

    def wait_idx(c, s):
      pltpu.make_async_copy(edge_hbm.at[sid * CHUNKS + c], s[0], s[2]).wait()

    def fire_gather(s):
      pltpu.async_copy(feat_hbm.at[s[0].at[0]], s[1], s[3])

    def wait_gather(s):
      pltpu.make_async_copy(feat_hbm.at[s[0].at[0]], s[1], s[3]).wait()

    def fire_scatter(s):
      pltpu.async_copy(s[1], acc_sh.at[s[0].at[1]], s[4], add=True)

    def wait_scatter(s):
      pltpu.make_async_copy(s[1], acc_sh.at[s[0].at[1]], s[4]).wait()

    def anchor(c, sk, drain=True, pf_idx=True, pf_gather=True):
      s = sets[sk]
      wait_gather(s)
      for i in range(K // 16):
        plsc.addupdate_scatter(degl, [s[0][1, pl.ds(i * 16, 16)]], ones16)
      fire_scatter(s)
      if pf_idx:
        y = sets[(sk + 2) % 4]
        if drain:
          wait_scatter(y)
        fire_idx(c + 2, y)
      if pf_gather:
        z = sets[(sk + 1) % 4]
        wait_idx(c + 1, z)
        fire_gather(z)

    fire_idx(0, sets[0])
    fire_idx(1, sets[1])
    wait_idx(0, sets[0])
    fire_gather(sets[0])
    anchor(0, 0, drain=False)
    anchor(1, 1, drain=False)

    def step(i, _):
      c0 = 4 * i + 2
      for k in range(4):
        anchor(c0 + k, (2 + k) % 4)
      return ()
    lax.fori_loop(0, (CHUNKS - 6) // 4, step, ())

    anchor(CHUNKS - 4, (CHUNKS - 4) % 4)
    anchor(CHUNKS - 3, (CHUNKS - 3) % 4)
    anchor(CHUNKS - 2, (CHUNKS - 2) % 4, pf_idx=False)
    anchor(CHUNKS - 1, (CHUNKS - 1) % 4, pf_idx=False, pf_gather=False)
    for k in range(4):
      wait_scatter(sets[(CHUNKS - 4 + k) % 4])

    pltpu.sync_copy(degl, deg_out.at[sid])

  @pl.when(cid == 0)
  def _():
    run(uf_hbm, ui_edge, item_deg)

  @pl.when(cid == 1)
  def _():
    run(if_hbm, iu_edge, user_deg)

  plsc.subcore_barrier()

  out0 = sid * OUT_STRIDE

  @pl.when(cid == 0)
  def _():
    pltpu.sync_copy(acc_sh.at[pl.ds(out0, 320)], item_sum.at[pl.ds(out0, 320)])

  @pl.when(cid == 1)
  def _():
    pltpu.sync_copy(acc_sh.at[pl.ds(out0, 320)], user_sum.at[pl.ds(out0, 320)])


def _tc_head_body(feat, msum, mdeg, ones_c, w_t, b, gamma, beta, out):
  d = lax.dot_general(mdeg[...], ones_c[...], (((0,), (0,)), ((), ())),
                      preferred_element_type=jnp.float32)
  r = 1.0 / jnp.maximum(d, 1.0)
  x = feat[...] + msum[...] * r
  h = jnp.dot(x, w_t[...], preferred_element_type=jnp.float32) + b[...]
  h = jnp.maximum(h, 0.0)
  mu = jnp.mean(h, axis=1, keepdims=True)
  var = jnp.mean((h - mu) ** 2, axis=1, keepdims=True)
  out[...] = (h - mu) * lax.rsqrt(var + EPS) * gamma[...] + beta[...]


def _tc_head(feat, msum, mdeg, ones_c, w_t, b, gamma, beta):
  bm = 1024
  grid = PAD // bm
  return pl.pallas_call(
      _tc_head_body,
      grid=(grid,),
      in_specs=[
          pl.BlockSpec((bm, DIM), lambda i: (i, 0)),
          pl.BlockSpec((bm, DIM), lambda i: (i, 0)),
          pl.BlockSpec((NS, bm), lambda i: (0, i)),
          pl.BlockSpec((NS, 1), lambda i: (0, 0)),
          pl.BlockSpec((DIM, DIM), lambda i: (0, 0)),
          pl.BlockSpec((1, DIM), lambda i: (0, 0)),
          pl.BlockSpec((1, DIM), lambda i: (0, 0)),
          pl.BlockSpec((1, DIM), lambda i: (0, 0)),
      ],
      out_specs=pl.BlockSpec((bm, DIM), lambda i: (i, 0)),
      out_shape=jax.ShapeDtypeStruct((N_NODES, DIM), jnp.float32),
  )(feat, msum, mdeg, ones_c, w_t, b, gamma, beta)


@jax.jit
def kernel(user_features, item_features, user_item_edge_index,
           item_user_edge_index, Wu, bu, Wi, bi, gamma, beta):
  def pack(edge_index):
    e = edge_index.astype(jnp.int32)
    return jnp.stack([e[0].reshape(-1, K), e[1].reshape(-1, K)], axis=1)

  ui_edge = pack(user_item_edge_index)
  iu_edge = pack(item_user_edge_index)
  zero_f = jnp.zeros((PAD // NS, DIM), jnp.float32)

  item_sum, item_deg, user_sum, user_deg = _sc_aggregate(
      user_features, item_features, ui_edge, iu_edge, zero_f)

  ones_c = jnp.ones((NS, 1), jnp.float32)
  g = gamma.reshape(1, DIM)
  bt = beta.reshape(1, DIM)
  user_new = _tc_head(user_features, user_sum, user_deg, ones_c,
                      Wu.T, bu.reshape(1, DIM), g, bt)
  item_new = _tc_head(item_features, item_sum, item_deg, ones_c,
                      Wi.T, bi.reshape(1, DIM), g, bt)
  return (user_new, item_new)


# speedup vs baseline: 1.4706x; 1.0812x over previous
import functools

import jax
import jax.numpy as jnp
from jax import lax
from jax.experimental import pallas as pl
from jax.experimental.pallas import tpu as pltpu
from jax.experimental.pallas import tpu_sc as plsc

N_NODES = 5000
N_EDGES = 320000
DIM = 128
EPS = 1e-5

NS = 16
PAD = 5120
K = 80
EDGES_PER_TILE = N_EDGES // NS
CHUNKS = EDGES_PER_TILE // K
OUT_STRIDE = 312

_mesh = plsc.VectorSubcoreMesh(core_axis_name="c", subcore_axis_name="s")


@functools.partial(
    pl.kernel,
    out_type=(
        jax.ShapeDtypeStruct((N_NODES, DIM), jnp.float32),
        jax.ShapeDtypeStruct((NS, PAD), jnp.float32),
        jax.ShapeDtypeStruct((N_NODES, DIM), jnp.float32),
        jax.ShapeDtypeStruct((NS, PAD), jnp.float32),
    ),
    mesh=_mesh,
    compiler_params=pltpu.CompilerParams(needs_layout_passes=False),
    scratch_types=[
        pltpu.VMEM_SHARED((PAD, DIM), jnp.float32),
    ] + [
        s
        for _ in range(4)
        for s in (pltpu.VMEM((K,), jnp.int32),
                  pltpu.VMEM((K,), jnp.int32),
                  pltpu.VMEM((K, DIM), jnp.float32),
                  pltpu.SemaphoreType.DMA,
                  pltpu.SemaphoreType.DMA,
                  pltpu.SemaphoreType.DMA)
    ] + [
        pltpu.VMEM((PAD,), jnp.float32),
    ],
)
def _sc_aggregate(uf_hbm, if_hbm, ui_src, ui_dst, iu_src, iu_dst, zero_f,
                  item_sum, item_deg, user_sum, user_deg,
                  acc_sh, *rest):
  sets = tuple(rest[6 * x:6 * x + 6] for x in range(4))
  degl = rest[24]
  cid = lax.axis_index("c")
  sid = lax.axis_index("s")
  row0 = sid * (PAD // NS)
  edge0 = sid * EDGES_PER_TILE
  zeros16 = jnp.zeros((16,), jnp.float32)
  ones16 = jnp.ones((16,), jnp.float32)

  pltpu.sync_copy(zero_f, acc_sh.at[pl.ds(row0, PAD // NS)])

  def zrow(i, _):
    degl[pl.ds(i * 16, 16)] = zeros16
    return ()
  lax.fori_loop(0, PAD // 16, zrow, ())
  plsc.subcore_barrier()

  def run(feat_hbm, src_hbm, dst_hbm, deg_out):
    def fire_idx(c, s):
      off = edge0 + c * K
      pltpu.async_copy(src_hbm.at[pl.ds(off, K)], s[0], s[3])
      pltpu.async_copy(dst_hbm.at[pl.ds(off, K)], s[1], s[3])

    def wait_idx(c, s):
      off = edge0 + c * K
      pltpu.make_async_copy(src_hbm.at[pl.ds(off, K)], s[0], s[3]).wait()
      pltpu.make_async_copy(dst_hbm.at[pl.ds(off, K)], s[1], s[3]).wait()

    def fire_gather(s):
      pltpu.async_copy(feat_hbm.at[s[0]], s[2], s[4])

    def wait_gather(s):
      pltpu.make_async_copy(feat_hbm.at[s[0]], s[2], s[4]).wait()

    def fire_scatter(s):
      pltpu.async_copy(s[2], acc_sh.at[s[1]], s[5], add=True)

    def wait_scatter(s):
      pltpu.make_async_copy(s[2], acc_sh.at[s[1]], s[5]).wait()

    def anchor(c, sk, drain=True, pf_idx=True, pf_gather=True):
      s = sets[sk]
      wait_gather(s)
      for i in range(K // 16):
        plsc.addupdate_scatter(degl, [s[1][pl.ds(i * 16, 16)]], ones16)
      fire_scatter(s)
      if pf_idx:
        y = sets[(sk + 2) % 4]
        if drain:
          wait_scatter(y)
        fire_idx(c + 2, y)
      if pf_gather:
        z = sets[(sk + 1) % 4]
        wait_idx(c + 1, z)
        fire_gather(z)

    fire_idx(0, sets[0])
    fire_idx(1, sets[1])
    wait_idx(0, sets[0])
    fire_gather(sets[0])
    anchor(0, 0, drain=False)
    anchor(1, 1, drain=False)

    def step(i, _):
      c0 = 4 * i + 2
      for k in range(4):
        anchor(c0 + k, (2 + k) % 4)
      return ()
    lax.fori_loop(0, (CHUNKS - 6) // 4, step, ())

    anchor(CHUNKS - 4, (CHUNKS - 4) % 4)
    anchor(CHUNKS - 3, (CHUNKS - 3) % 4)
    anchor(CHUNKS - 2, (CHUNKS - 2) % 4, pf_idx=False)
    anchor(CHUNKS - 1, (CHUNKS - 1) % 4, pf_idx=False, pf_gather=False)
    for k in range(4):
      wait_scatter(sets[(CHUNKS - 4 + k) % 4])

    pltpu.sync_copy(degl, deg_out.at[sid])

  @pl.when(cid == 0)
  def _():
    run(uf_hbm, ui_src, ui_dst, item_deg)

  @pl.when(cid == 1)
  def _():
    run(if_hbm, iu_src, iu_dst, user_deg)

  plsc.subcore_barrier()

  out0 = sid * OUT_STRIDE

  @pl.when(cid == 0)
  def _():
    pltpu.sync_copy(acc_sh.at[pl.ds(out0, 320)], item_sum.at[pl.ds(out0, 320)])

  @pl.when(cid == 1)
  def _():
    pltpu.sync_copy(acc_sh.at[pl.ds(out0, 320)], user_sum.at[pl.ds(out0, 320)])


def _tc_head_body(feat, msum, mdeg, ones_c, w_t, b, gamma, beta, out):
  d = lax.dot_general(mdeg[...], ones_c[...], (((0,), (0,)), ((), ())),
                      preferred_element_type=jnp.float32)
  r = 1.0 / jnp.maximum(d, 1.0)
  x = feat[...] + msum[...] * r
  h = jnp.dot(x, w_t[...], preferred_element_type=jnp.float32) + b[...]
  h = jnp.maximum(h, 0.0)
  mu = jnp.mean(h, axis=1, keepdims=True)
  var = jnp.mean((h - mu) ** 2, axis=1, keepdims=True)
  out[...] = (h - mu) * lax.rsqrt(var + EPS) * gamma[...] + beta[...]


def _tc_head(feat, msum, mdeg, ones_c, w_t, b, gamma, beta):
  bm = 1024
  grid = PAD // bm
  return pl.pallas_call(
      _tc_head_body,
      grid=(grid,),
      in_specs=[
          pl.BlockSpec((bm, DIM), lambda i: (i, 0)),
          pl.BlockSpec((bm, DIM), lambda i: (i, 0)),
          pl.BlockSpec((NS, bm), lambda i: (0, i)),
          pl.BlockSpec((NS, 1), lambda i: (0, 0)),
          pl.BlockSpec((DIM, DIM), lambda i: (0, 0)),
          pl.BlockSpec((1, DIM), lambda i: (0, 0)),
          pl.BlockSpec((1, DIM), lambda i: (0, 0)),
          pl.BlockSpec((1, DIM), lambda i: (0, 0)),
      ],
      out_specs=pl.BlockSpec((bm, DIM), lambda i: (i, 0)),
      out_shape=jax.ShapeDtypeStruct((N_NODES, DIM), jnp.float32),
  )(feat, msum, mdeg, ones_c, w_t, b, gamma, beta)


@jax.jit
def kernel(user_features, item_features, user_item_edge_index,
           item_user_edge_index, Wu, bu, Wi, bi, gamma, beta):
  ui_src = user_item_edge_index[0].astype(jnp.int32)
  ui_dst = user_item_edge_index[1].astype(jnp.int32)
  iu_src = item_user_edge_index[0].astype(jnp.int32)
  iu_dst = item_user_edge_index[1].astype(jnp.int32)
  zero_f = jnp.zeros((PAD // NS, DIM), jnp.float32)

  item_sum, item_deg, user_sum, user_deg = _sc_aggregate(
      user_features, item_features, ui_src, ui_dst, iu_src, iu_dst, zero_f)

  ones_c = jnp.ones((NS, 1), jnp.float32)
  g = gamma.reshape(1, DIM)
  bt = beta.reshape(1, DIM)
  user_new = _tc_head(user_features, user_sum, user_deg, ones_c,
                      Wu.T, bu.reshape(1, DIM), g, bt)
  item_new = _tc_head(item_features, item_sum, item_deg, ones_c,
                      Wi.T, bi.reshape(1, DIM), g, bt)
  return (user_new, item_new)
